# Initial kernel scaffold; baseline (speedup 1.0000x reference)
import jax
import jax.numpy as jnp
from jax.experimental import pallas as pl


def kernel(input_features, word_table, tt_table, singleton_weight, singleton_bias):
    raise NotImplementedError("write your pallas kernel here")


# speedup vs baseline: 1.3715x; 1.3715x over previous
import functools

import jax
import jax.numpy as jnp
from jax import lax
from jax.experimental import pallas as pl
from jax.experimental.pallas import tpu as pltpu
from jax.experimental.pallas import tpu_sc as plsc

_HID = 64
_B, _S = 1024, 200
_TOT = _B * _S
_NC, _NS = 2, 16
_NW = _NC * _NS
_CHUNK = 128
_PER_W = _TOT // _NW
_NCH = _PER_W // _CHUNK
_NBUF = 5
_NG = _NCH // _NBUF
_LANE = 16


def _sc_embed_body(ids_hbm, cw_hbm, table_hbm, out_hbm,
                   idx_v, gbuf, sbuf, cw_v, gsem, ssem):
    wid = lax.axis_index("s") * _NC + lax.axis_index("c")
    row_base = wid * _PER_W

    pltpu.sync_copy(ids_hbm.at[wid], idx_v)
    pltpu.sync_copy(cw_hbm, cw_v)

    w_vec = cw_v[pl.ds(_HID, _LANE)]
    c_vecs = [cw_v[pl.ds(_LANE * j, _LANE)] for j in range(_HID // _LANE)]

    def gather_start(k, b):
        pltpu.make_async_copy(
            table_hbm.at[idx_v.at[k]], gbuf.at[b], gsem.at[b]).start()

    def gather_wait(b):
        pltpu.make_async_copy(
            table_hbm.at[idx_v.at[0]], gbuf.at[b], gsem.at[b]).wait()

    def store_start(k, b):
        pltpu.make_async_copy(
            sbuf.at[b],
            out_hbm.at[pl.ds(row_base + k * _CHUNK, _CHUNK)],
            ssem.at[b]).start()

    def store_wait(b):
        pltpu.make_async_copy(
            sbuf.at[b],
            out_hbm.at[pl.ds(row_base, _CHUNK)],
            ssem.at[b]).wait()

    def fma(b):
        gb = gbuf.at[b]
        sb = sbuf.at[b]

        def body(i, carry):
            r0 = i * 4
            for dr in range(4):
                r = r0 + dr
                for j in range(_HID // _LANE):
                    v = gb[r, pl.ds(_LANE * j, _LANE)]
                    sb[r, pl.ds(_LANE * j, _LANE)] = v * w_vec + c_vecs[j]
            return carry

        lax.fori_loop(0, _CHUNK // 4, body, 0, unroll=False)

    for b in range(_NBUF):
        gather_start(b, b)

    def outer(g, carry):
        for b in range(_NBUF):
            k = g * _NBUF + b
            gather_wait(b)

            @pl.when(g > 0)
            def _wait_prev_store():
                store_wait(b)

            fma(b)
            store_start(k, b)

            @pl.when(g < _NG - 1)
            def _refill():
                gather_start(k + _NBUF, b)
        return carry

    lax.fori_loop(0, _NG, outer, 0, unroll=False)

    for b in range(_NBUF):
        store_wait(b)


_embed_call = functools.partial(
    pl.kernel,
    out_type=jax.ShapeDtypeStruct((_TOT, _HID), jnp.float32),
    mesh=plsc.VectorSubcoreMesh(core_axis_name="c", subcore_axis_name="s"),
    compiler_params=pltpu.CompilerParams(use_tc_tiling_on_sc=False),
    scratch_types=[
        pltpu.VMEM((_NCH, _CHUNK), jnp.int32),
        pltpu.VMEM((_NBUF, _CHUNK, _HID), jnp.float32),
        pltpu.VMEM((_NBUF, _CHUNK, _HID), jnp.float32),
        pltpu.VMEM((_HID + _LANE,), jnp.float32),
        pltpu.SemaphoreType.DMA((_NBUF,)),
        pltpu.SemaphoreType.DMA((_NBUF,)),
    ],
)(_sc_embed_body)


def kernel(input_features, word_table, tt_table, singleton_weight, singleton_bias):
    ids = input_features.reshape(_NW, _NCH, _CHUNK).astype(jnp.int32)
    w = singleton_weight[0].astype(jnp.float32)
    c = tt_table[0].astype(jnp.float32) * w + singleton_bias[0].astype(jnp.float32)
    cw = jnp.concatenate([c, jnp.full((_LANE,), w, jnp.float32)])
    out = _embed_call(ids, cw, word_table.astype(jnp.float32))
    return out.reshape(_B, _S, _HID)
